# Initial kernel scaffold; baseline (speedup 1.0000x reference)
import jax
import jax.numpy as jnp
from jax.experimental import pallas as pl


def kernel(feature, pos, W_pool, W_regress, W_extract):
    raise NotImplementedError("write your pallas kernel here")


# speedup vs baseline: 9.2552x; 9.2552x over previous
import jax
import jax.numpy as jnp
from jax.experimental import pallas as pl

_BS = 16
_NPG = 4096
_C = 256
_NKP = 64
_KNN = 10


def _bdot(a, b):
    return jnp.dot(a.astype(jnp.bfloat16), b.astype(jnp.bfloat16),
                   preferred_element_type=jnp.float32)


def _body(feat_ref, pos_ref, pos_t_ref, wr_ref, wef_ref, wep_ref, out_ref):
    feat = feat_ref[...].astype(jnp.bfloat16)
    pos = pos_ref[...]
    s = jnp.dot(feat, wr_ref[...].astype(jnp.bfloat16),
                preferred_element_type=jnp.float32)
    m = jnp.max(s, axis=0, keepdims=True)
    p = jnp.exp(s - m)
    pw = p / jnp.sum(p, axis=0, keepdims=True)
    kp = jax.lax.dot_general(pw.astype(jnp.bfloat16), pos.astype(jnp.bfloat16),
                             (((0,), (0,)), ((), ())),
                             preferred_element_type=jnp.float32)

    d2 = jnp.zeros((_NKP, _NPG), jnp.float32)
    for d in range(3):
        diff = kp[:, d:d + 1] - pos_t_ref[0, d:d + 1, :]
        d2 = d2 + diff * diff

    iota = jax.lax.broadcasted_iota(jnp.int32, (_NKP, _NPG), 1)
    acc = jnp.zeros((_NKP, _C), jnp.float32)
    wef = wef_ref[...].astype(jnp.bfloat16)
    wep = wep_ref[...].astype(jnp.bfloat16)
    big = jnp.float32(jnp.inf)
    zero = jnp.zeros((), jnp.float32)
    for _ in range(_KNN):
        dmin = jnp.min(d2, axis=1, keepdims=True)
        sel = d2 <= dmin
        idx = jnp.min(jnp.where(sel, iota, _NPG), axis=1, keepdims=True)
        hit = iota == idx
        fj = jnp.dot(hit.astype(jnp.bfloat16), feat,
                     preferred_element_type=jnp.float32)
        pj = jnp.concatenate(
            [jnp.sum(jnp.where(hit, pos_t_ref[0, d:d + 1, :], zero),
                     axis=1, keepdims=True) for d in range(3)], axis=1)
        ext = _bdot(fj, wef) + _bdot(pj - kp, wep)
        acc += jnp.maximum(ext, 0.0)
        d2 = jnp.where(hit, big, d2)
    out_ref[0] = acc * (1.0 / _KNN)


def kernel(feature, pos, W_pool, W_regress, W_extract):
    del W_pool
    wr = W_regress[:_C]
    wef = W_extract[:_C]
    wep = W_extract[_C:]
    pos_t = pos.reshape(_BS, _NPG, 3).transpose(0, 2, 1)
    return pl.pallas_call(
        _body,
        grid=(_BS,),
        in_specs=[
            pl.BlockSpec((_NPG, _C), lambda b: (b, 0)),
            pl.BlockSpec((_NPG, 3), lambda b: (b, 0)),
            pl.BlockSpec((1, 3, _NPG), lambda b: (b, 0, 0)),
            pl.BlockSpec((_C, _NKP), lambda b: (0, 0)),
            pl.BlockSpec((_C, _C), lambda b: (0, 0)),
            pl.BlockSpec((3, _C), lambda b: (0, 0)),
        ],
        out_specs=pl.BlockSpec((1, _NKP, _C), lambda b: (b, 0, 0)),
        out_shape=jax.ShapeDtypeStruct((_BS, _NKP, _C), jnp.float32),
    )(feature, pos, pos_t, wr, wef, wep)


# speedup vs baseline: 9.6493x; 1.0426x over previous
import jax
import jax.numpy as jnp
from jax.experimental import pallas as pl

_BS = 16
_NPG = 4096
_C = 256
_NKP = 64
_KNN = 10


def _body(fp_ref, pos_t_ref, wr_ref, wef_ref, wep_ref, out_ref):
    fp = fp_ref[...]
    feat = fp[:, :_C]
    posb = fp[:, _C:]
    s = jnp.dot(feat, wr_ref[...], preferred_element_type=jnp.float32)
    m = jnp.max(s, axis=0, keepdims=True)
    p = jnp.exp(s - m)
    pw = p / jnp.sum(p, axis=0, keepdims=True)
    kp = jax.lax.dot_general(pw.astype(jnp.bfloat16), posb,
                             (((0,), (0,)), ((), ())),
                             preferred_element_type=jnp.float32)

    d2 = jnp.zeros((_NKP, _NPG), jnp.float32)
    for d in range(3):
        diff = kp[:, d:d + 1] - pos_t_ref[0, d:d + 1, :]
        d2 = d2 + diff * diff

    iota = jax.lax.broadcasted_iota(jnp.int32, (_NKP, _NPG), 1)
    big = jnp.float32(jnp.inf)
    hits = []
    for _ in range(_KNN):
        dmin = jnp.min(d2, axis=1, keepdims=True)
        idx = jnp.min(jnp.where(d2 <= dmin, iota, _NPG), axis=1, keepdims=True)
        hit = iota == idx
        hits.append(hit.astype(jnp.bfloat16))
        d2 = jnp.where(hit, big, d2)

    onehot = jnp.concatenate(hits, axis=0)
    g = jnp.dot(onehot, fp, preferred_element_type=jnp.float32)
    fj = g[:, :_C].astype(jnp.bfloat16)
    pj = g[:, _C:]
    kp_rep = jnp.concatenate([kp] * _KNN, axis=0)
    rel = (pj - kp_rep).astype(jnp.bfloat16)
    ext = (jnp.dot(fj, wef_ref[...], preferred_element_type=jnp.float32)
           + jnp.dot(rel, wep_ref[...], preferred_element_type=jnp.float32))
    ext = jnp.maximum(ext, 0.0).reshape(_KNN, _NKP, _C)
    out_ref[0] = jnp.sum(ext, axis=0) * (1.0 / _KNN)


def kernel(feature, pos, W_pool, W_regress, W_extract):
    del W_pool
    bf = jnp.bfloat16
    fp = jnp.concatenate([feature, pos], axis=1).astype(bf)
    wr = W_regress[:_C].astype(bf)
    wef = W_extract[:_C].astype(bf)
    wep = W_extract[_C:].astype(bf)
    pos_t = pos.reshape(_BS, _NPG, 3).transpose(0, 2, 1)
    return pl.pallas_call(
        _body,
        grid=(_BS,),
        in_specs=[
            pl.BlockSpec((_NPG, _C + 3), lambda b: (b, 0)),
            pl.BlockSpec((1, 3, _NPG), lambda b: (b, 0, 0)),
            pl.BlockSpec((_C, _NKP), lambda b: (0, 0)),
            pl.BlockSpec((_C, _C), lambda b: (0, 0)),
            pl.BlockSpec((3, _C), lambda b: (0, 0)),
        ],
        out_specs=pl.BlockSpec((1, _NKP, _C), lambda b: (b, 0, 0)),
        out_shape=jax.ShapeDtypeStruct((_BS, _NKP, _C), jnp.float32),
    )(fp, pos_t, wr, wef, wep)


# speedup vs baseline: 14.7796x; 1.5317x over previous
import jax
import jax.numpy as jnp
from jax.experimental import pallas as pl

_BS = 16
_NPG = 4096
_C = 256
_NKP = 64
_KNN = 10


def _body(feat_ref, posb_ref, pos_t_ref, wr_ref, wef_ref, wep_ref, out_ref):
    feat = feat_ref[...].astype(jnp.bfloat16)
    posb = posb_ref[...]
    s = jnp.dot(feat, wr_ref[...], preferred_element_type=jnp.float32)
    m = jnp.max(s, axis=0, keepdims=True)
    p = jnp.exp(s - m)
    pw = p / jnp.sum(p, axis=0, keepdims=True)
    kp = jax.lax.dot_general(pw.astype(jnp.bfloat16), posb,
                             (((0,), (0,)), ((), ())),
                             preferred_element_type=jnp.float32)

    d2 = jnp.zeros((_NKP, _NPG), jnp.float32)
    for d in range(3):
        diff = kp[:, d:d + 1] - pos_t_ref[0, d:d + 1, :]
        d2 = d2 + diff * diff

    iotaf = jax.lax.broadcasted_iota(jnp.int32, (_NKP, _NPG), 1).astype(
        jnp.float32)
    big = jnp.float32(jnp.inf)
    npgf = jnp.float32(_NPG)
    hits = []
    for _ in range(_KNN):
        dmin = jnp.min(d2, axis=1, keepdims=True)
        idx = jnp.min(jnp.where(d2 <= dmin, iotaf, npgf), axis=1, keepdims=True)
        hit = iotaf == idx
        hits.append(hit.astype(jnp.bfloat16))
        d2 = jnp.where(hit, big, d2)

    onehot = jnp.concatenate(hits, axis=0)
    g = jnp.dot(onehot, feat, preferred_element_type=jnp.float32)
    pj = jnp.dot(onehot, posb, preferred_element_type=jnp.float32)
    fj = g.astype(jnp.bfloat16)
    kp_rep = jnp.concatenate([kp] * _KNN, axis=0)
    rel = (pj - kp_rep).astype(jnp.bfloat16)
    ext = (jnp.dot(fj, wef_ref[...], preferred_element_type=jnp.float32)
           + jnp.dot(rel, wep_ref[...], preferred_element_type=jnp.float32))
    ext = jnp.maximum(ext, 0.0).reshape(_KNN, _NKP, _C)
    out_ref[0] = jnp.sum(ext, axis=0) * (1.0 / _KNN)


def kernel(feature, pos, W_pool, W_regress, W_extract):
    del W_pool
    bf = jnp.bfloat16
    posb = pos.astype(bf)
    wr = W_regress[:_C].astype(bf)
    wef = W_extract[:_C].astype(bf)
    wep = W_extract[_C:].astype(bf)
    pos_t = pos.reshape(_BS, _NPG, 3).transpose(0, 2, 1)
    return pl.pallas_call(
        _body,
        grid=(_BS,),
        in_specs=[
            pl.BlockSpec((_NPG, _C), lambda b: (b, 0)),
            pl.BlockSpec((_NPG, 3), lambda b: (b, 0)),
            pl.BlockSpec((1, 3, _NPG), lambda b: (b, 0, 0)),
            pl.BlockSpec((_C, _NKP), lambda b: (0, 0)),
            pl.BlockSpec((_C, _C), lambda b: (0, 0)),
            pl.BlockSpec((3, _C), lambda b: (0, 0)),
        ],
        out_specs=pl.BlockSpec((1, _NKP, _C), lambda b: (b, 0, 0)),
        out_shape=jax.ShapeDtypeStruct((_BS, _NKP, _C), jnp.float32),
    )(feature, posb, pos_t, wr, wef, wep)
